# Initial kernel scaffold; baseline (speedup 1.0000x reference)
import jax
import jax.numpy as jnp
from jax.experimental import pallas as pl


def kernel(pos_indices, pos_emb_table):
    raise NotImplementedError("write your pallas kernel here")


# speedup vs baseline: 1.6346x; 1.6346x over previous
import functools

import jax
import jax.numpy as jnp
from jax import lax
from jax.experimental import pallas as pl
from jax.experimental.pallas import tpu as pltpu
from jax.experimental.pallas import tpu_sc as plsc

NUM_POS = 17
EMB_DIM = 10
BATCH = 16384

NUM_CORES = 2
NUM_SUBCORES = 16
NUM_WORKERS = NUM_CORES * NUM_SUBCORES
B_PER_W = BATCH // NUM_WORKERS
LANES = 16
OUT_PER_W = B_PER_W * EMB_DIM
GROUPS = B_PER_W // LANES

_MESH = plsc.VectorSubcoreMesh(core_axis_name="c", subcore_axis_name="s")


@functools.partial(
    pl.kernel,
    out_type=jax.ShapeDtypeStruct((BATCH * EMB_DIM,), jnp.float32),
    mesh=_MESH,
    scratch_types=[
        pltpu.VMEM((B_PER_W,), jnp.int32),
        pltpu.VMEM((NUM_POS, EMB_DIM), jnp.float32),
        pltpu.VMEM((OUT_PER_W,), jnp.float32),
    ],
    compiler_params=pltpu.CompilerParams(
        use_tc_tiling_on_sc=False, needs_layout_passes=False),
)
def _emb_lookup(idx_hbm, table_hbm, out_hbm, idx_v, table_v, out_v):
    wid = lax.axis_index("s") * NUM_CORES + lax.axis_index("c")
    base = wid * B_PER_W
    pltpu.sync_copy(idx_hbm.at[pl.ds(base, B_PER_W)], idx_v)
    pltpu.sync_copy(table_hbm, table_v)
    lane = lax.iota(jnp.int32, LANES)
    rpat = []
    cpat = []
    for k in range(EMB_DIM):
        p = lane + (k * LANES)
        r = lax.shift_right_logical(p * 6554, 16)
        rpat.append(r)
        cpat.append(p - r * EMB_DIM)
    for g in range(GROUPS):
        for k in range(EMB_DIM):
            rows = plsc.load_gather(idx_v, [rpat[k] + g * LANES])
            vals = plsc.load_gather(table_v, [rows, cpat[k]])
            out_v[pl.ds(g * EMB_DIM * LANES + k * LANES, LANES)] = vals
    pltpu.sync_copy(out_v, out_hbm.at[pl.ds(wid * OUT_PER_W, OUT_PER_W)])


def kernel(pos_indices, pos_emb_table):
    flat = _emb_lookup(pos_indices.astype(jnp.int32),
                       pos_emb_table.astype(jnp.float32))
    return flat.reshape(BATCH, EMB_DIM)
